# baseline (speedup 1.0000x reference)
import functools

import jax
import jax.numpy as jnp
import numpy as np
from jax import lax
from jax.experimental import pallas as pl
from jax.experimental.pallas import tpu as pltpu
from jax.experimental.pallas import tpu_sc as plsc

LOG2_T = 19
T = 1 << LOG2_T
N_LEVELS = 16
BASE_RES = 16
PER_LEVEL_SCALE = 1.3819
N_PTS = 524288
MASK = (1 << LOG2_T) - 1
P1 = np.int32(np.uint32(2654435761).astype(np.int64) - (1 << 32))

RES = [int(np.floor(BASE_RES * PER_LEVEL_SCALE**l)) for l in range(N_LEVELS)]
PLANES = ((0, 1), (0, 2), (1, 2))

NC, NS = 2, 16
NW = NC * NS
PPW = N_PTS // NW
B = 512
NGRP = B // 16
NIB = B // 128
NCHUNK = PPW // B

F = 2
OUTD = N_LEVELS * F


def _body(x, t0, t1, t2, out, xv, idxv, wv, rowsv, outv, sem):
    tabs = (t0, t1, t2)
    wid = lax.axis_index("s") * NC + lax.axis_index("c")
    base_w = wid * PPW
    iota = lax.iota(jnp.int32, 16)

    @pl.loop(0, NCHUNK)
    def _chunk(ci):
        base = base_w + ci * B
        pltpu.sync_copy(x.at[pl.ds(base, B), :], xv)

        @pl.loop(0, N_LEVELS)
        def _level(l):
            res_f = jnp.float32(RES[0])
            for ll in range(1, N_LEVELS):
                res_f = jnp.where(l == ll, jnp.float32(RES[ll]), res_f)

            @pl.loop(0, NGRP)
            def _grp_a(g):
                o16 = g * 16
                i0 = [None] * 3
                i1 = [None] * 3
                y0 = [None] * 3
                y1 = [None] * 3
                pvec = iota + o16
                for c in range(3):
                    cvec = jnp.full((16,), 1, jnp.int32) * c
                    xs = plsc.load_gather(xv, [pvec, cvec]) * res_f
                    ic = xs.astype(jnp.int32)
                    wv[pl.ds(c * B + o16, 16)] = xs - ic.astype(jnp.float32)
                    i0[c] = ic
                    i1[c] = ic + 1
                for c in (1, 2):
                    y0[c] = i0[c] * P1
                    y1[c] = y0[c] + P1
                j = 0
                for (a, b) in PLANES:
                    for (xi, ym) in ((i0[a], y0[b]), (i1[a], y0[b]),
                                     (i0[a], y1[b]), (i1[a], y1[b])):
                        idxv[pl.ds(j * B + o16, 16)] = (xi ^ ym) & MASK
                        j += 1

            handles = []
            for j in range(12):
                tp = tabs[j // 4]
                for k in range(NIB):
                    handles.append(pltpu.async_copy(
                        tp.at[l].at[idxv.at[pl.ds(j * B + k * 128, 128)]],
                        rowsv.at[pl.ds(j * B + k * 128, 128), :], sem))
            for h in handles:
                h.wait()

            @pl.loop(0, NGRP)
            def _grp_b(g):
                o16 = g * 16
                pvec = iota + o16
                w = [wv[pl.ds(c * B + o16, 16)] for c in range(3)]
                prod = [None, None]
                for p, (a, b) in enumerate(PLANES):
                    wx = w[a]
                    wy = w[b]
                    for f in range(2):
                        fvec = jnp.full((16,), f, jnp.int32)
                        cv = []
                        for cnr in range(4):
                            rbase = (p * 4 + cnr) * B
                            cv.append(plsc.load_gather(
                                rowsv, [pvec + rbase, fvec]))
                        lo = cv[0] + wx * (cv[1] - cv[0])
                        hi = cv[2] + wx * (cv[3] - cv[2])
                        fv = lo + wy * (hi - lo)
                        prod[f] = fv if p == 0 else prod[f] * fv
                col0 = l * 2
                for f in range(2):
                    colvec = jnp.full((16,), 1, jnp.int32) * (col0 + f)
                    plsc.store_scatter(outv, [pvec, colvec], prod[f])

        pltpu.sync_copy(outv, out.at[pl.ds(base, B)])


_mesh = plsc.VectorSubcoreMesh(
    core_axis_name="c", subcore_axis_name="s", num_cores=NC, num_subcores=NS)

_sc_call = functools.partial(
    pl.kernel,
    out_type=jax.ShapeDtypeStruct((N_PTS, OUTD), jnp.float32),
    mesh=_mesh,
    compiler_params=pltpu.CompilerParams(
        needs_layout_passes=False, use_tc_tiling_on_sc=False),
    scratch_types=[
        pltpu.VMEM((B, 3), jnp.float32),
        pltpu.VMEM((12 * B,), jnp.int32),
        pltpu.VMEM((3 * B,), jnp.float32),
        pltpu.VMEM((12 * B, 2), jnp.float32),
        pltpu.VMEM((B, OUTD), jnp.float32),
        pltpu.SemaphoreType.DMA,
    ],
)(_body)


@jax.jit
def kernel(x, table0, table1, table2):
    return _sc_call(x, table0, table1, table2)


# speedup vs baseline: 3.4040x; 3.4040x over previous
import functools

import jax
import jax.numpy as jnp
import numpy as np
from jax import lax
from jax.experimental import pallas as pl
from jax.experimental.pallas import tpu as pltpu
from jax.experimental.pallas import tpu_sc as plsc

LOG2_T = 19
T = 1 << LOG2_T
N_LEVELS = 16
BASE_RES = 16
PER_LEVEL_SCALE = 1.3819
N_PTS = 524288
MASK = (1 << LOG2_T) - 1
P1 = np.int32(np.uint32(2654435761).astype(np.int64) - (1 << 32))

RES = [int(np.floor(BASE_RES * PER_LEVEL_SCALE**l)) for l in range(N_LEVELS)]
PLANES = ((0, 1), (0, 2), (1, 2))

NC, NS = 2, 16
NW = NC * NS
PPW = N_PTS // NW
B = 512
NGRP = B // 16
NIB = B // 128
NCHUNK = PPW // B

F = 2
OUTD = N_LEVELS * F


def _body(x, t0, t1, t2, out, xv, idxv, wv, rowsv, outv, sem):
    tabs = (t0, t1, t2)
    wid = lax.axis_index("s") * NC + lax.axis_index("c")
    base_w = wid * PPW
    iota = lax.iota(jnp.int32, 16)

    @pl.loop(0, NCHUNK)
    def _chunk(ci):
        base = base_w + ci * B
        pltpu.sync_copy(x.at[pl.ds(base, B), :], xv)

        @pl.loop(0, N_LEVELS)
        def _level(l):
            res_f = jnp.float32(RES[0])
            for ll in range(1, N_LEVELS):
                res_f = jnp.where(l == ll, jnp.float32(RES[ll]), res_f)

            @pl.loop(0, NGRP)
            def _grp_a(g):
                o16 = g * 16
                i0 = [None] * 3
                i1 = [None] * 3
                y0 = [None] * 3
                y1 = [None] * 3
                pvec = iota + o16
                for c in range(3):
                    cvec = jnp.full((16,), 1, jnp.int32) * c
                    xs = plsc.load_gather(xv, [pvec, cvec]) * res_f
                    ic = xs.astype(jnp.int32)
                    wv[pl.ds(c * B + o16, 16)] = xs - ic.astype(jnp.float32)
                    i0[c] = ic
                    i1[c] = ic + 1
                for c in (1, 2):
                    y0[c] = i0[c] * P1
                    y1[c] = y0[c] + P1
                j = 0
                for (a, b) in PLANES:
                    for (xi, ym) in ((i0[a], y0[b]), (i1[a], y0[b]),
                                     (i0[a], y1[b]), (i1[a], y1[b])):
                        idxv[pl.ds(j * B + o16, 16)] = (xi ^ ym) & MASK
                        j += 1

            handles = []
            for j in range(12):
                tp = tabs[j // 4]
                for k in range(NIB):
                    handles.append(pltpu.async_copy(
                        tp.at[l].at[idxv.at[pl.ds(j * B + k * 128, 128)]],
                        rowsv.at[pl.ds(j * B + k * 128, 128), :], sem))
            for h in handles:
                h.wait()

            @pl.loop(0, NGRP)
            def _grp_b(g):
                o16 = g * 16
                pvec = iota + o16
                w = [wv[pl.ds(c * B + o16, 16)] for c in range(3)]
                prod = [None, None]
                for p, (a, b) in enumerate(PLANES):
                    wx = w[a]
                    wy = w[b]
                    for f in range(2):
                        fvec = jnp.full((16,), f, jnp.int32)
                        cv = []
                        for cnr in range(4):
                            rbase = (p * 4 + cnr) * B
                            cv.append(plsc.load_gather(
                                rowsv, [pvec + rbase, fvec]))
                        lo = cv[0] + wx * (cv[1] - cv[0])
                        hi = cv[2] + wx * (cv[3] - cv[2])
                        fv = lo + wy * (hi - lo)
                        prod[f] = fv if p == 0 else prod[f] * fv
                col0 = l * 2
                for f in range(2):
                    colvec = jnp.full((16,), 1, jnp.int32) * (col0 + f)
                    plsc.store_scatter(outv, [pvec, colvec], prod[f])

        pltpu.sync_copy(outv, out.at[pl.ds(base, B)])


_mesh = plsc.VectorSubcoreMesh(
    core_axis_name="c", subcore_axis_name="s", num_cores=NC, num_subcores=NS)

_sc_call = functools.partial(
    pl.kernel,
    out_type=jax.ShapeDtypeStruct((N_PTS, OUTD), jnp.float32),
    mesh=_mesh,
    compiler_params=pltpu.CompilerParams(
        needs_layout_passes=False, use_tc_tiling_on_sc=False),
    scratch_types=[
        pltpu.VMEM((B, 3), jnp.float32),
        pltpu.VMEM((12 * B,), jnp.int32),
        pltpu.VMEM((3 * B,), jnp.float32),
        pltpu.VMEM((12 * B, 2), jnp.float32),
        pltpu.VMEM((B, OUTD), jnp.float32),
        pltpu.SemaphoreType.DMA,
    ],
)(_body)


@jax.jit
def kernel(x, table0, table1, table2):
    from jax.experimental import layout as _layout
    lin = _layout.Layout((0, 1, 2), tiling=())
    t0, t1, t2 = _layout.with_layout_constraint(
        (table0, table1, table2), (lin, lin, lin))
    return _sc_call(x, t0, t1, t2)
